# baseline (speedup 1.0000x reference)
import functools

import jax
import jax.numpy as jnp
from jax import lax
from jax.experimental import pallas as pl
from jax.experimental.pallas import tpu as pltpu
from jax.experimental.pallas import tpu_sc as plsc

_POOL = 1024
_SEL = 4
_PLEN = 8
_DIM = 768
_BATCH = 1024


def _select_body(q_ref, k_ref, f_ref, mrows_ref, mosts_ref, match_ref):
    q = q_ref[...]
    kp = k_ref[...]
    f = f_ref[...]
    dot = lax.dot_general(q, kp, (((1,), (1,)), ((), ())),
                          preferred_element_type=jnp.float32)
    qn = jnp.sqrt(jnp.sum(q * q, axis=1, keepdims=True))
    kn = jnp.sqrt(jnp.sum(kp * kp, axis=1))
    denom = jnp.maximum(qn * kn.reshape(1, _POOL), 1e-8)
    match = dot / denom
    match_ref[...] = match
    inv = 1.0 / f
    w = inv / jnp.sum(jnp.abs(inv))

    colid = lax.broadcasted_iota(jnp.int32, (_BATCH, _POOL), 1)
    a = match * w.reshape(1, _POOL)
    selmask = jnp.zeros((_BATCH, _POOL), jnp.bool_)
    neg = jnp.float32(-jnp.inf)
    for s in range(_SEL):
        m = jnp.max(a, axis=1, keepdims=True)
        idx = jnp.min(jnp.where(a >= m, colid, _POOL), axis=1,
                      keepdims=True)
        hit = colid == idx
        selmask = selmask | hit
        if s + 1 < _SEL:
            a = jnp.where(hit, neg, a)
    counts = jnp.sum(selmask.astype(jnp.int32), axis=0)

    c = counts.reshape(1, _POOL)
    pid = lax.broadcasted_iota(jnp.int32, (1, _POOL), 1)
    lane16 = lax.broadcasted_iota(jnp.int32, (1, 16), 1)
    mvec = jnp.zeros((1, 16), jnp.int32)
    for s in range(_SEL):
        mx = jnp.max(c)
        midx = jnp.min(jnp.where(c == mx, pid, _POOL))
        c = jnp.where(pid == midx, jnp.int32(-1), c)
        mvec = jnp.where(lane16 == s, midx, mvec)
        mrows_ref[pl.ds(s, 1), :] = match_ref[pl.ds(midx, 1), :]
    mosts_ref[...] = mvec.reshape(16)


_stage_select = pl.pallas_call(
    _select_body,
    out_shape=(
        jax.ShapeDtypeStruct((_SEL, _POOL), jnp.float32),
        jax.ShapeDtypeStruct((16,), jnp.int32),
    ),
    scratch_shapes=[pltpu.VMEM((_BATCH, _POOL), jnp.float32)],
)


@functools.cache
def _make_bcast():
    info = plsc.get_sparse_core_info()
    nc, ns = info.num_cores, info.num_subcores
    rows_per = _BATCH // (nc * ns)
    mesh = plsc.VectorSubcoreMesh(core_axis_name="c", subcore_axis_name="s")

    @functools.partial(
        pl.kernel,
        mesh=mesh,
        out_type=[
            jax.ShapeDtypeStruct((_BATCH, _SEL, _PLEN, _DIM), jnp.float32),
        ],
        scratch_types=[
            pltpu.VMEM((_SEL,), jnp.int32),
            pltpu.VMEM((_SEL, _PLEN, _DIM), jnp.float32),
            pltpu.SemaphoreType.DMA,
        ],
    )
    def _bcast(mosts_hbm, prompt_hbm, psel_hbm, idx_v, prows_v, sem):
        wid = lax.axis_index("s") * nc + lax.axis_index("c")
        base = wid * rows_per
        pltpu.sync_copy(mosts_hbm.at[pl.ds(0, _SEL)], idx_v)
        pltpu.async_copy(prompt_hbm.at[idx_v], prows_v, sem).wait()
        pending = []
        for b in range(rows_per):
            pending.append(
                pltpu.async_copy(prows_v, psel_hbm.at[base + b], sem))
            while len(pending) > 8:
                pending.pop(0).wait()
        for d in pending:
            d.wait()

    return _bcast


_TILE_B = 128


def _tile_body(mrows_ref, out_ref):
    out_ref[...] = jnp.broadcast_to(mrows_ref[...][None],
                                    (_TILE_B, _SEL, _POOL))


_tile_mrows = pl.pallas_call(
    _tile_body,
    grid=(_BATCH // _TILE_B,),
    in_specs=[pl.BlockSpec((_SEL, _POOL), lambda i: (0, 0))],
    out_specs=pl.BlockSpec((_TILE_B, _SEL, _POOL), lambda i: (i, 0, 0)),
    out_shape=jax.ShapeDtypeStruct((_BATCH, _SEL, _POOL), jnp.float32),
)


def kernel(query, key_param, prompt, frequency):
    mrows, mosts16 = _stage_select(query, key_param, frequency)
    (prompt_sel,) = _make_bcast()(mosts16, prompt)
    match_sel = _tile_mrows(mrows)
    return match_sel, prompt_sel


# speedup vs baseline: 1.0248x; 1.0248x over previous
import functools

import jax
import jax.numpy as jnp
from jax import lax
from jax.experimental import pallas as pl
from jax.experimental.pallas import tpu as pltpu
from jax.experimental.pallas import tpu_sc as plsc

_POOL = 1024
_SEL = 4
_PLEN = 8
_DIM = 768
_BATCH = 1024


def _select_body(q_ref, k_ref, f_ref, mrows_ref, mosts_ref, match_ref):
    q = q_ref[...]
    kp = k_ref[...]
    f = f_ref[...]
    dot = lax.dot_general(q, kp, (((1,), (1,)), ((), ())),
                          preferred_element_type=jnp.float32)
    qn = jnp.sqrt(jnp.sum(q * q, axis=1, keepdims=True))
    kn = jnp.sqrt(jnp.sum(kp * kp, axis=1))
    denom = jnp.maximum(qn * kn.reshape(1, _POOL), 1e-8)
    match = dot / denom
    match_ref[...] = match
    inv = 1.0 / f
    w = inv / jnp.sum(jnp.abs(inv))

    colid = lax.broadcasted_iota(jnp.int32, (_BATCH, _POOL), 1)
    a = match * w.reshape(1, _POOL)
    selmask = jnp.zeros((_BATCH, _POOL), jnp.bool_)
    neg = jnp.float32(-jnp.inf)
    for s in range(_SEL):
        idx = jnp.argmax(a, axis=1).astype(jnp.int32)
        hit = colid == idx[:, None]
        selmask = selmask | hit
        if s + 1 < _SEL:
            a = jnp.where(hit, neg, a)
    counts = jnp.sum(selmask.astype(jnp.int32), axis=0)

    c = counts.reshape(1, _POOL)
    pid = lax.broadcasted_iota(jnp.int32, (1, _POOL), 1)
    lane16 = lax.broadcasted_iota(jnp.int32, (1, 16), 1)
    mvec = jnp.zeros((1, 16), jnp.int32)
    for s in range(_SEL):
        mx = jnp.max(c)
        midx = jnp.min(jnp.where(c == mx, pid, _POOL))
        c = jnp.where(pid == midx, jnp.int32(-1), c)
        mvec = jnp.where(lane16 == s, midx, mvec)
        mrows_ref[pl.ds(s, 1), :] = match_ref[pl.ds(midx, 1), :]
    mosts_ref[...] = mvec.reshape(16)


_stage_select = pl.pallas_call(
    _select_body,
    out_shape=(
        jax.ShapeDtypeStruct((_SEL, _POOL), jnp.float32),
        jax.ShapeDtypeStruct((16,), jnp.int32),
    ),
    scratch_shapes=[pltpu.VMEM((_BATCH, _POOL), jnp.float32)],
)


@functools.cache
def _make_bcast():
    info = plsc.get_sparse_core_info()
    nc, ns = info.num_cores, info.num_subcores
    rows_per = _BATCH // (nc * ns)
    mesh = plsc.VectorSubcoreMesh(core_axis_name="c", subcore_axis_name="s")

    @functools.partial(
        pl.kernel,
        mesh=mesh,
        out_type=[
            jax.ShapeDtypeStruct((_BATCH, _SEL, _PLEN, _DIM), jnp.float32),
        ],
        scratch_types=[
            pltpu.VMEM((_SEL,), jnp.int32),
            pltpu.VMEM((_SEL, _PLEN, _DIM), jnp.float32),
            pltpu.SemaphoreType.DMA,
        ],
    )
    def _bcast(mosts_hbm, prompt_hbm, psel_hbm, idx_v, prows_v, sem):
        wid = lax.axis_index("s") * nc + lax.axis_index("c")
        base = wid * rows_per
        pltpu.sync_copy(mosts_hbm.at[pl.ds(0, _SEL)], idx_v)
        pltpu.async_copy(prompt_hbm.at[idx_v], prows_v, sem).wait()
        pending = []
        for b in range(rows_per):
            pending.append(
                pltpu.async_copy(prows_v, psel_hbm.at[base + b], sem))
            while len(pending) > 8:
                pending.pop(0).wait()
        for d in pending:
            d.wait()

    return _bcast


_TILE_B = 128


def _tile_body(mrows_ref, out_ref):
    out_ref[...] = jnp.broadcast_to(mrows_ref[...][None],
                                    (_TILE_B, _SEL, _POOL))


_tile_mrows = pl.pallas_call(
    _tile_body,
    grid=(_BATCH // _TILE_B,),
    in_specs=[pl.BlockSpec((_SEL, _POOL), lambda i: (0, 0))],
    out_specs=pl.BlockSpec((_TILE_B, _SEL, _POOL), lambda i: (i, 0, 0)),
    out_shape=jax.ShapeDtypeStruct((_BATCH, _SEL, _POOL), jnp.float32),
)


def kernel(query, key_param, prompt, frequency):
    mrows, mosts16 = _stage_select(query, key_param, frequency)
    (prompt_sel,) = _make_bcast()(mosts16, prompt)
    match_sel = _tile_mrows(mrows)
    return match_sel, prompt_sel
